# baseline (speedup 1.0000x reference)
import functools

import jax
import jax.numpy as jnp
from jax import lax
from jax.experimental import pallas as pl
from jax.experimental.pallas import tpu as pltpu
from jax.experimental.pallas import tpu_sc as plsc

VOCAB = 100000
N_POS = 8192
D = 1024
B = 16
TOTAL = 32768

NC = 2
NS = 16
L = 16
NW = NC * NS
TOK_W = TOTAL // NW
CH = 16
NCHUNK = TOK_W // CH
IDX_ROWS = TOK_W // L


def _body(ids_hbm, cu_hbm, wte_hbm, wpe_hbm, out_hbm,
          idx_v, pos_v, cu_v, a0, a1, b0, b1, sg0, sg1, so0, so1):
  cid = lax.axis_index("c")
  sid = lax.axis_index("s")
  wid = sid * NC + cid
  tokbase = wid * TOK_W

  pltpu.sync_copy(ids_hbm.at[pl.ds(wid * IDX_ROWS, IDX_ROWS)], idx_v)
  pltpu.sync_copy(cu_hbm, cu_v)

  cuvec = cu_v[:]
  cbs = [cuvec.at[jnp.full((L,), j, jnp.int32)].get(mode="promise_in_bounds")
         for j in range(1, B)]
  iota = lax.iota(jnp.int32, L)

  def pos_body(i, carry):
    tok = tokbase + i * L + iota
    m = jnp.zeros((L,), jnp.int32)
    for cb in cbs:
      m = jnp.maximum(m, jnp.where(cb <= tok, cb, jnp.int32(0)))
    pos_v[i, :] = tok - m
    return carry

  lax.fori_loop(0, IDX_ROWS, pos_body, 0)

  def start_gather(ch, a, b, sg):
    pltpu.make_async_copy(wte_hbm.at[idx_v.at[ch]], a, sg).start()
    pltpu.make_async_copy(wpe_hbm.at[pos_v.at[ch]], b, sg).start()

  def wait_gather(a, b, sg):
    pltpu.make_async_copy(wte_hbm.at[pl.ds(0, CH)], a, sg).wait()
    pltpu.make_async_copy(wte_hbm.at[pl.ds(0, CH)], b, sg).wait()

  def do_add(a, b):
    def add_body(k, carry):
      for r in range(CH):
        sl = pl.ds(k * L, L)
        a[r, sl] = a[r, sl] + b[r, sl]
      return carry
    lax.fori_loop(0, D // L, add_body, 0)

  def start_store(ch, a, so):
    dst = out_hbm.at[pl.ds(tokbase + ch * CH, CH)]
    pltpu.make_async_copy(a, dst, so).start()

  def wait_store(a, so):
    pltpu.make_async_copy(a, out_hbm.at[pl.ds(0, CH)], so).wait()

  bufs = ((a0, b0, sg0, so0), (a1, b1, sg1, so1))

  start_gather(0, a0, b0, sg0)
  start_gather(1, a1, b1, sg1)
  wait_gather(a0, b0, sg0)
  do_add(a0, b0)
  start_store(0, a0, so0)

  def main_body(j, carry):
    for ph in (1, 0):
      ch = 2 * j + 1 + (1 - ph)
      a, b, sg, so = bufs[ph]
      an, bn, sgn, son = bufs[1 - ph]
      wait_store(an, son)
      start_gather(ch + 1, an, bn, sgn)
      wait_gather(a, b, sg)
      do_add(a, b)
      start_store(ch, a, so)
    return carry

  lax.fori_loop(0, (NCHUNK - 2) // 2, main_body, 0)

  wait_store(a0, so0)
  wait_gather(a1, b1, sg1)
  do_add(a1, b1)
  start_store(NCHUNK - 1, a1, so1)
  wait_store(a1, so1)


LSEG = TOTAL // B
CHU = 32
PPW = 2 * CHU
HPW = 2
SEGS = B


def _body_uniform(ids_hbm, wte_hbm, wpe_hbm, out_hbm,
                  idx_v, wc, a0, a1, sg0, sg1, so0, so1):
  cid = lax.axis_index("c")
  sid = lax.axis_index("s")
  wid = sid * NC + cid
  p0 = wid * PPW

  for h in range(HPW):
    for s in range(SEGS):
      src = ids_hbm.at[pl.ds(s * (LSEG // CHU) + wid * HPW + h, 1)]
      pltpu.sync_copy(src, idx_v.at[pl.ds(h * SEGS + s, 1)])

  bufs = ((a0, sg0, so0), (a1, sg1, so1))

  def start_gather(c, ph):
    a, sg, _ = bufs[ph]
    pltpu.make_async_copy(wte_hbm.at[idx_v.at[c]], a, sg).start()

  def wait_gather(ph):
    a, sg, _ = bufs[ph]
    pltpu.make_async_copy(wte_hbm.at[pl.ds(0, CHU)], a, sg).wait()

  def do_add(ph):
    a, _, _ = bufs[ph]
    def add_body(k, carry):
      for r in range(CHU):
        sl = pl.ds(k * L, L)
        a[r, sl] = a[r, sl] + wc[r, sl]
      return carry
    lax.fori_loop(0, D // L, add_body, 0)

  def start_store(s, h, ph):
    a, _, so = bufs[ph]
    base = s * LSEG + p0 + h * CHU
    pltpu.make_async_copy(a, out_hbm.at[pl.ds(base, CHU)], so).start()

  def wait_store(ph):
    a, _, so = bufs[ph]
    pltpu.make_async_copy(a, out_hbm.at[pl.ds(0, CHU)], so).wait()

  for h in range(HPW):
    c0 = h * SEGS
    start_gather(c0, 0)
    pltpu.sync_copy(wpe_hbm.at[pl.ds(p0 + h * CHU, CHU)], wc)

    start_gather(c0 + 1, 1)
    wait_gather(0)
    do_add(0)
    start_store(0, h, 0)

    def pair_body(i, carry):
      for u in (0, 1):
        s = 2 * i + 1 + u
        ph = 1 - u
        wait_store(1 - ph)
        start_gather(c0 + s + 1, 1 - ph)
        wait_gather(ph)
        do_add(ph)
        start_store(s, h, ph)
      return carry

    lax.fori_loop(0, (SEGS - 2) // 2, pair_body, 0)

    wait_store(0)
    wait_gather(1)
    do_add(1)
    start_store(SEGS - 1, h, 1)
    wait_store(1)


@functools.partial(jax.jit, static_argnames=())
def kernel(packed_input_ids, cu_seqlens, wte, wpe):
  ids2d = packed_input_ids.reshape(TOTAL // L, L)
  cu16 = cu_seqlens[:B].astype(jnp.int32)
  mesh = plsc.VectorSubcoreMesh(core_axis_name="c", subcore_axis_name="s")
  out_type = jax.ShapeDtypeStruct((TOTAL, D), jnp.float32)
  k_gen = pl.kernel(
      _body,
      out_type=out_type,
      mesh=mesh,
      scratch_types=[
          pltpu.VMEM((IDX_ROWS, L), jnp.int32),
          pltpu.VMEM((IDX_ROWS, L), jnp.int32),
          pltpu.VMEM((B,), jnp.int32),
          pltpu.VMEM((CH, D), jnp.float32),
          pltpu.VMEM((CH, D), jnp.float32),
          pltpu.VMEM((CH, D), jnp.float32),
          pltpu.VMEM((CH, D), jnp.float32),
          pltpu.SemaphoreType.DMA,
          pltpu.SemaphoreType.DMA,
          pltpu.SemaphoreType.DMA,
          pltpu.SemaphoreType.DMA,
      ],
  )
  k_uni = pl.kernel(
      _body_uniform,
      out_type=out_type,
      mesh=mesh,
      scratch_types=[
          pltpu.VMEM((HPW * SEGS, CHU), jnp.int32),
          pltpu.VMEM((CHU, D), jnp.float32),
          pltpu.VMEM((CHU, D), jnp.float32),
          pltpu.VMEM((CHU, D), jnp.float32),
          pltpu.SemaphoreType.DMA,
          pltpu.SemaphoreType.DMA,
          pltpu.SemaphoreType.DMA,
          pltpu.SemaphoreType.DMA,
      ],
  )
  expected = jnp.arange(B + 1, dtype=jnp.int32) * LSEG
  is_uniform = jnp.all(cu_seqlens.astype(jnp.int32) == expected)
  return lax.cond(is_uniform,
                  lambda: k_uni(packed_input_ids.reshape(TOTAL // CHU, CHU),
                                wte, wpe),
                  lambda: k_gen(ids2d, cu16, wte, wpe))


# speedup vs baseline: 1.2069x; 1.2069x over previous
import functools

import jax
import jax.numpy as jnp
from jax import lax
from jax.experimental import pallas as pl
from jax.experimental.pallas import tpu as pltpu
from jax.experimental.pallas import tpu_sc as plsc

VOCAB = 100000
N_POS = 8192
D = 1024
B = 16
TOTAL = 32768

NC = 2
NS = 16
L = 16
NW = NC * NS
TOK_W = TOTAL // NW
CH = 16
NCHUNK = TOK_W // CH
IDX_ROWS = TOK_W // L


def _body(ids_hbm, cu_hbm, wte_hbm, wpe_hbm, out_hbm,
          idx_v, pos_v, cu_v, a0, a1, b0, b1, sg0, sg1, so0, so1):
  cid = lax.axis_index("c")
  sid = lax.axis_index("s")
  wid = sid * NC + cid
  tokbase = wid * TOK_W

  pltpu.sync_copy(ids_hbm.at[pl.ds(wid * IDX_ROWS, IDX_ROWS)], idx_v)
  pltpu.sync_copy(cu_hbm, cu_v)

  cuvec = cu_v[:]
  cbs = [cuvec.at[jnp.full((L,), j, jnp.int32)].get(mode="promise_in_bounds")
         for j in range(1, B)]
  iota = lax.iota(jnp.int32, L)

  def pos_body(i, carry):
    tok = tokbase + i * L + iota
    m = jnp.zeros((L,), jnp.int32)
    for cb in cbs:
      m = jnp.maximum(m, jnp.where(cb <= tok, cb, jnp.int32(0)))
    pos_v[i, :] = tok - m
    return carry

  lax.fori_loop(0, IDX_ROWS, pos_body, 0)

  def start_gather(ch, a, b, sg):
    pltpu.make_async_copy(wte_hbm.at[idx_v.at[ch]], a, sg).start()
    pltpu.make_async_copy(wpe_hbm.at[pos_v.at[ch]], b, sg).start()

  def wait_gather(a, b, sg):
    pltpu.make_async_copy(wte_hbm.at[pl.ds(0, CH)], a, sg).wait()
    pltpu.make_async_copy(wte_hbm.at[pl.ds(0, CH)], b, sg).wait()

  def do_add(a, b):
    def add_body(k, carry):
      for r in range(CH):
        sl = pl.ds(k * L, L)
        a[r, sl] = a[r, sl] + b[r, sl]
      return carry
    lax.fori_loop(0, D // L, add_body, 0)

  def start_store(ch, a, so):
    dst = out_hbm.at[pl.ds(tokbase + ch * CH, CH)]
    pltpu.make_async_copy(a, dst, so).start()

  def wait_store(a, so):
    pltpu.make_async_copy(a, out_hbm.at[pl.ds(0, CH)], so).wait()

  bufs = ((a0, b0, sg0, so0), (a1, b1, sg1, so1))

  start_gather(0, a0, b0, sg0)
  start_gather(1, a1, b1, sg1)
  wait_gather(a0, b0, sg0)
  do_add(a0, b0)
  start_store(0, a0, so0)

  def main_body(j, carry):
    for ph in (1, 0):
      ch = 2 * j + 1 + (1 - ph)
      a, b, sg, so = bufs[ph]
      an, bn, sgn, son = bufs[1 - ph]
      wait_store(an, son)
      start_gather(ch + 1, an, bn, sgn)
      wait_gather(a, b, sg)
      do_add(a, b)
      start_store(ch, a, so)
    return carry

  lax.fori_loop(0, (NCHUNK - 2) // 2, main_body, 0)

  wait_store(a0, so0)
  wait_gather(a1, b1, sg1)
  do_add(a1, b1)
  start_store(NCHUNK - 1, a1, so1)
  wait_store(a1, so1)


LSEG = TOTAL // B
PPW = 64
HPW = 2
HALF = PPW // HPW
CHU = 16
SEGS = B
CPH = SEGS * (HALF // CHU)


def _body_uniform(ids_hbm, wte_hbm, wpe_hbm, out_hbm,
                  idx_v, wc, a0, a1, o0, o1, sg0, sg1, so0, so1):
  cid = lax.axis_index("c")
  sid = lax.axis_index("s")
  wid = sid * NC + cid
  p0 = wid * PPW

  for h in range(HPW):
    for s in range(SEGS):
      row = s * (LSEG // L) + wid * (PPW // L) + h * 2
      src = ids_hbm.at[pl.ds(row, 2)]
      pltpu.sync_copy(src, idx_v.at[pl.ds(h * CPH + s * 2, 2)])

  abufs = (a0, a1)
  obufs = (o0, o1)
  gsems = (sg0, sg1)
  osems = (so0, so1)

  def start_gather(c, ph):
    pltpu.make_async_copy(wte_hbm.at[idx_v.at[c]], abufs[ph], gsems[ph]).start()

  def wait_gather(ph):
    pltpu.make_async_copy(wte_hbm.at[pl.ds(0, CHU)], abufs[ph], gsems[ph]).wait()

  def do_add(ph):
    a, o = abufs[ph], obufs[ph]
    def add_body(k, carry):
      for r in range(CHU):
        sl = pl.ds(k * L, L)
        o[r, sl] = a[r, sl] + wc[ph * CHU + r, sl]
      return carry
    lax.fori_loop(0, D // L, add_body, 0)

  def start_store(s, q, h, ph):
    base = s * LSEG + p0 + h * HALF + q * CHU
    pltpu.make_async_copy(obufs[ph], out_hbm.at[pl.ds(base, CHU)],
                          osems[ph]).start()

  def wait_store(ph):
    pltpu.make_async_copy(obufs[ph], out_hbm.at[pl.ds(0, CHU)],
                          osems[ph]).wait()

  def chunk(c0, j, s, q, h, do_wait_store, do_start_next):
    ph = q
    wait_gather(ph)
    if do_wait_store:
      wait_store(ph)
    do_add(ph)
    if do_start_next:
      start_gather(c0 + j + 2, ph)
    start_store(s, q, h, ph)

  for h in range(HPW):
    c0 = h * CPH
    start_gather(c0, 0)
    start_gather(c0 + 1, 1)
    pltpu.sync_copy(wpe_hbm.at[pl.ds(p0 + h * HALF, HALF)], wc)

    chunk(c0, 0, 0, 0, h, False, True)
    chunk(c0, 1, 0, 1, h, False, True)

    def pair_body(i, carry):
      s = i + 1
      chunk(c0, 2 * i + 2, s, 0, h, True, True)
      chunk(c0, 2 * i + 3, s, 1, h, True, True)
      return carry

    lax.fori_loop(0, SEGS - 2, pair_body, 0)

    chunk(c0, CPH - 2, SEGS - 1, 0, h, True, False)
    chunk(c0, CPH - 1, SEGS - 1, 1, h, True, False)
    wait_store(0)
    wait_store(1)


@functools.partial(jax.jit, static_argnames=())
def kernel(packed_input_ids, cu_seqlens, wte, wpe):
  ids2d = packed_input_ids.reshape(TOTAL // L, L)
  cu16 = cu_seqlens[:B].astype(jnp.int32)
  mesh = plsc.VectorSubcoreMesh(core_axis_name="c", subcore_axis_name="s")
  out_type = jax.ShapeDtypeStruct((TOTAL, D), jnp.float32)
  k_gen = pl.kernel(
      _body,
      out_type=out_type,
      mesh=mesh,
      scratch_types=[
          pltpu.VMEM((IDX_ROWS, L), jnp.int32),
          pltpu.VMEM((IDX_ROWS, L), jnp.int32),
          pltpu.VMEM((B,), jnp.int32),
          pltpu.VMEM((CH, D), jnp.float32),
          pltpu.VMEM((CH, D), jnp.float32),
          pltpu.VMEM((CH, D), jnp.float32),
          pltpu.VMEM((CH, D), jnp.float32),
          pltpu.SemaphoreType.DMA,
          pltpu.SemaphoreType.DMA,
          pltpu.SemaphoreType.DMA,
          pltpu.SemaphoreType.DMA,
      ],
  )
  k_uni = pl.kernel(
      _body_uniform,
      out_type=out_type,
      mesh=mesh,
      scratch_types=[
          pltpu.VMEM((HPW * CPH, L), jnp.int32),
          pltpu.VMEM((HALF, D), jnp.float32),
          pltpu.VMEM((CHU, D), jnp.float32),
          pltpu.VMEM((CHU, D), jnp.float32),
          pltpu.VMEM((CHU, D), jnp.float32),
          pltpu.VMEM((CHU, D), jnp.float32),
          pltpu.SemaphoreType.DMA,
          pltpu.SemaphoreType.DMA,
          pltpu.SemaphoreType.DMA,
          pltpu.SemaphoreType.DMA,
      ],
  )
  expected = jnp.arange(B + 1, dtype=jnp.int32) * LSEG
  is_uniform = jnp.all(cu_seqlens.astype(jnp.int32) == expected)
  return lax.cond(is_uniform,
                  lambda: k_uni(ids2d, wte, wpe),
                  lambda: k_gen(ids2d, cu16, wte, wpe))
